# baseline (speedup 1.0000x reference)
import functools

import jax
import jax.numpy as jnp
from jax import lax
from jax.experimental import pallas as pl
from jax.experimental.pallas import tpu as pltpu
from jax.experimental.pallas import tpu_sc as plsc

NC = 2
NS = 16
CH = 80
NB = 3


def _segsum_body(with_counts, nb, ch, n_nodes, n_feat, e_per_w, n_chunks, big,
                 *refs):
  if with_counts:
    (x_hbm, src_hbm, dst_hbm, zrow_hbm, ones_hbm, p_out, c_out) = refs[:7]
    rest = refs[7:]
  else:
    (x_hbm, src_hbm, dst_hbm, zrow_hbm, p_out) = refs[:5]
    ones_hbm = c_out = None
    rest = refs[5:]
  src_all, dst_all = rest[0], rest[1]
  rows = rest[2:2 + nb]
  acc = rest[2 + nb]
  gsem = rest[3 + nb:3 + 2 * nb]
  ssem = rest[3 + 2 * nb:3 + 3 * nb]
  ones_v = rows[0]

  c = lax.axis_index("c")
  s = lax.axis_index("s")
  wid = s * NC + c
  tail = n_nodes - NS * big

  def zero_acc():
    pltpu.sync_copy(zrow_hbm, acc.at[pl.ds(s * big, big)])

    @pl.when(s == 0)
    def _():
      pltpu.sync_copy(zrow_hbm.at[pl.ds(0, tail)],
                      acc.at[pl.ds(NS * big, tail)])

  def write_out(dst_arr):
    pltpu.sync_copy(acc.at[pl.ds(s * big, big)],
                    dst_arr.at[c, pl.ds(s * big, big)])

    @pl.when(s == 0)
    def _():
      pltpu.sync_copy(acc.at[pl.ds(NS * big, tail)],
                      dst_arr.at[c, pl.ds(NS * big, tail)])

  ebase = wid * e_per_w
  pltpu.sync_copy(src_hbm.at[pl.ds(ebase, e_per_w)], src_all)
  pltpu.sync_copy(dst_hbm.at[pl.ds(ebase, e_per_w)], dst_all)

  if with_counts:
    zero_acc()
    pltpu.sync_copy(ones_hbm, ones_v)
    plsc.subcore_barrier()

    def cdesc(j, b):
      return pltpu.make_async_copy(
          ones_v, acc.at[dst_all.at[pl.ds(j * ch, ch)]], ssem[b])

    for j in range(nb):
      cdesc(j, j).start(add=True)

    tc = (n_chunks - nb) // nb

    @pl.loop(nb, nb + tc * nb, step=nb)
    def _(j0):
      for k2 in range(nb):
        j = j0 + k2
        cdesc(j - nb, k2).wait()
        cdesc(j, k2).start(add=True)

    for j in range(nb + tc * nb, n_chunks):
      cdesc(j - nb, j % nb).wait()
      cdesc(j, j % nb).start(add=True)

    for db in range(nb):
      j = n_chunks - nb + db
      cdesc(j, j % nb).wait()

    plsc.subcore_barrier()
    write_out(c_out)
    plsc.subcore_barrier()

  zero_acc()
  plsc.subcore_barrier()

  def gdesc(j, b):
    return pltpu.make_async_copy(
        x_hbm.at[src_all.at[pl.ds(j * ch, ch)]], rows[b], gsem[b])

  def sdesc(j, b):
    return pltpu.make_async_copy(
        rows[b], acc.at[dst_all.at[pl.ds(j * ch, ch)]], ssem[b])

  P = nb
  gdesc(0, 0).start()
  for j in range(P):
    if j + 1 >= nb:
      sdesc(j + 1 - nb, (j + 1) % nb).wait()
    gdesc(j + 1, (j + 1) % nb).start()
    gdesc(j, j % nb).wait()
    sdesc(j, j % nb).start(add=True)

  tr = (n_chunks - 1 - P) // nb

  @pl.loop(P, P + tr * nb, step=nb)
  def _(j0):
    for k2 in range(nb):
      j = j0 + k2
      b = (P + k2) % nb
      bn = (P + k2 + 1) % nb
      sdesc(j + 1 - nb, bn).wait()
      gdesc(j + 1, bn).start()
      gdesc(j, b).wait()
      sdesc(j, b).start(add=True)

  for j in range(P + tr * nb, n_chunks):
    if j + 1 < n_chunks:
      sdesc(j + 1 - nb, (j + 1) % nb).wait()
      gdesc(j + 1, (j + 1) % nb).start()
    gdesc(j, j % nb).wait()
    sdesc(j, j % nb).start(add=True)

  for db in range(nb):
    j = n_chunks - nb + db
    sdesc(j, j % nb).wait()

  plsc.subcore_barrier()
  write_out(p_out)


@functools.partial(jax.jit, static_argnums=(3,))
def _segsum(x, src, dst, with_counts):
  n_nodes, n_feat = x.shape
  n_edges = src.shape[0]
  nw = NC * NS
  e_per_w = n_edges // nw
  nb = NB
  ch = CH
  n_chunks = e_per_w // ch
  assert e_per_w * nw == n_edges and n_chunks * ch == e_per_w
  assert n_chunks >= 2 * nb + 2
  big = (n_nodes // NS) // 8 * 8
  tail = n_nodes - NS * big
  assert 0 < tail <= big

  mesh = plsc.VectorSubcoreMesh(core_axis_name="c", subcore_axis_name="s")
  out_type = [jax.ShapeDtypeStruct((NC, n_nodes, n_feat), jnp.float32)]
  inputs = [x, src, dst, jnp.zeros((big, n_feat), jnp.float32)]
  scratch = [
      pltpu.VMEM((e_per_w,), jnp.int32),
      pltpu.VMEM((e_per_w,), jnp.int32),
  ] + [pltpu.VMEM((ch, n_feat), jnp.float32) for _ in range(nb)]
  if with_counts:
    out_type.append(jax.ShapeDtypeStruct((NC, n_nodes, n_feat), jnp.float32))
    inputs.append(jnp.ones((ch, n_feat), jnp.float32))
  scratch.append(pltpu.VMEM_SHARED((n_nodes, n_feat), jnp.float32))
  scratch += [pltpu.SemaphoreType.DMA for _ in range(2 * nb)]

  body = functools.partial(_segsum_body, with_counts, nb, ch, n_nodes, n_feat,
                           e_per_w, n_chunks, big)
  fn = pl.kernel(body, out_type=out_type, mesh=mesh, scratch_types=scratch)
  return fn(*inputs)


def _dense_body(relu, p_ref, c_ref, x_ref, wl_ref, b_ref, wr_ref, o_ref):
  ssum = p_ref[0] + p_ref[1]
  cnt = c_ref[0][:, 0:1] + c_ref[1][:, 0:1]
  mean = ssum / jnp.maximum(cnt, 1.0)
  acc = lax.dot_general(mean, wl_ref[...], (((1,), (1,)), ((), ())),
                        preferred_element_type=jnp.float32)
  acc = acc + lax.dot_general(x_ref[...], wr_ref[...], (((1,), (1,)), ((), ())),
                              preferred_element_type=jnp.float32)
  acc = acc + b_ref[...]
  o_ref[...] = jnp.maximum(acc, 0.0) if relu else acc


def _dense(p, cpart, x, wl, bl, wr, relu):
  n, f = x.shape
  blk = 1000
  grid = (n // blk,)
  body = functools.partial(_dense_body, relu)
  return pl.pallas_call(
      body,
      grid=grid,
      in_specs=[
          pl.BlockSpec((NC, blk, f), lambda i: (0, i, 0)),
          pl.BlockSpec((NC, blk, f), lambda i: (0, i, 0)),
          pl.BlockSpec((blk, f), lambda i: (i, 0)),
          pl.BlockSpec((f, f), lambda i: (0, 0)),
          pl.BlockSpec((1, f), lambda i: (0, 0)),
          pl.BlockSpec((f, f), lambda i: (0, 0)),
      ],
      out_specs=pl.BlockSpec((blk, f), lambda i: (i, 0)),
      out_shape=jax.ShapeDtypeStruct((n, f), jnp.float32),
  )(p, cpart, x, wl, bl.reshape(1, f), wr)


def kernel(x, edge_index, lin1_W, lin1_b, c1_Wl, c1_bl, c1_Wr,
           c2_Wl, c2_bl, c2_Wr):
  e32 = edge_index.astype(jnp.int32)
  src = e32[0]
  dst = e32[1]

  p1, cpart = _segsum(x, src, dst, True)
  h = _dense(p1, cpart, x, c1_Wl, c1_bl, c1_Wr, True)
  (p2,) = _segsum(h, src, dst, False)
  out = _dense(p2, cpart, h, c2_Wl, c2_bl, c2_Wr, False)
  return out


# speedup vs baseline: 1.0092x; 1.0092x over previous
import functools

import jax
import jax.numpy as jnp
from jax import lax
from jax.experimental import pallas as pl
from jax.experimental.pallas import tpu as pltpu
from jax.experimental.pallas import tpu_sc as plsc

NC = 2
NS = 16
CH = 80
NB = 3


def _segsum_body(with_counts, nb, ch, n_nodes, n_feat, e_per_w, n_chunks, big,
                 *refs):
  if with_counts:
    (x_hbm, src_hbm, dst_hbm, zrow_hbm, ones_hbm, p_out, c_out) = refs[:7]
    rest = refs[7:]
  else:
    (x_hbm, src_hbm, dst_hbm, zrow_hbm, p_out) = refs[:5]
    ones_hbm = c_out = None
    rest = refs[5:]
  src_all, dst_all = rest[0], rest[1]
  rows = rest[2:2 + nb]
  acc = rest[2 + nb]
  gsem = rest[3 + nb:3 + 2 * nb]
  ssem = rest[3 + 2 * nb:3 + 3 * nb]
  ones_v = rows[0]

  c = lax.axis_index("c")
  s = lax.axis_index("s")
  wid = s * NC + c
  tail = n_nodes - NS * big

  def zero_acc():
    pltpu.sync_copy(zrow_hbm, acc.at[pl.ds(s * big, big)])

    @pl.when(s == 0)
    def _():
      pltpu.sync_copy(zrow_hbm.at[pl.ds(0, tail)],
                      acc.at[pl.ds(NS * big, tail)])

  def write_out(dst_arr):
    pltpu.sync_copy(acc.at[pl.ds(s * big, big)],
                    dst_arr.at[c, pl.ds(s * big, big)])

    @pl.when(s == 0)
    def _():
      pltpu.sync_copy(acc.at[pl.ds(NS * big, tail)],
                      dst_arr.at[c, pl.ds(NS * big, tail)])

  ebase = wid * e_per_w
  pltpu.sync_copy(src_hbm.at[pl.ds(ebase, e_per_w)], src_all)
  pltpu.sync_copy(dst_hbm.at[pl.ds(ebase, e_per_w)], dst_all)

  if with_counts:
    zero_acc()
    pltpu.sync_copy(ones_hbm, ones_v)
    plsc.subcore_barrier()

    def cdesc(j, b):
      return pltpu.make_async_copy(
          ones_v, acc.at[dst_all.at[pl.ds(j * ch, ch)]], ssem[b])

    for j in range(nb):
      cdesc(j, j).start(add=True)

    tc = (n_chunks - nb) // nb

    @pl.loop(nb, nb + tc * nb, step=nb)
    def _(j0):
      for k2 in range(nb):
        j = j0 + k2
        cdesc(j - nb, k2).wait()
        cdesc(j, k2).start(add=True)

    for j in range(nb + tc * nb, n_chunks):
      cdesc(j - nb, j % nb).wait()
      cdesc(j, j % nb).start(add=True)

    for db in range(nb):
      j = n_chunks - nb + db
      cdesc(j, j % nb).wait()

    plsc.subcore_barrier()
    write_out(c_out)
    plsc.subcore_barrier()

  zero_acc()
  plsc.subcore_barrier()

  hh = ch // 2

  class gdesc:
    def __init__(self, j, b):
      self.d = [
          pltpu.make_async_copy(
              x_hbm.at[src_all.at[pl.ds(j * ch + h * hh, hh)]],
              rows[b].at[pl.ds(h * hh, hh)], gsem[b])
          for h in (0, 1)
      ]

    def start(self):
      for d in self.d:
        d.start()

    def wait(self):
      for d in self.d:
        d.wait()

  def sdesc(j, b):
    return pltpu.make_async_copy(
        rows[b], acc.at[dst_all.at[pl.ds(j * ch, ch)]], ssem[b])

  P = nb
  gdesc(0, 0).start()
  for j in range(P):
    if j + 1 >= nb:
      sdesc(j + 1 - nb, (j + 1) % nb).wait()
    gdesc(j + 1, (j + 1) % nb).start()
    gdesc(j, j % nb).wait()
    sdesc(j, j % nb).start(add=True)

  tr = (n_chunks - 1 - P) // nb

  @pl.loop(P, P + tr * nb, step=nb)
  def _(j0):
    for k2 in range(nb):
      j = j0 + k2
      b = (P + k2) % nb
      bn = (P + k2 + 1) % nb
      sdesc(j + 1 - nb, bn).wait()
      gdesc(j + 1, bn).start()
      gdesc(j, b).wait()
      sdesc(j, b).start(add=True)

  for j in range(P + tr * nb, n_chunks):
    if j + 1 < n_chunks:
      sdesc(j + 1 - nb, (j + 1) % nb).wait()
      gdesc(j + 1, (j + 1) % nb).start()
    gdesc(j, j % nb).wait()
    sdesc(j, j % nb).start(add=True)

  for db in range(nb):
    j = n_chunks - nb + db
    sdesc(j, j % nb).wait()

  plsc.subcore_barrier()
  write_out(p_out)


@functools.partial(jax.jit, static_argnums=(3,))
def _segsum(x, src, dst, with_counts):
  n_nodes, n_feat = x.shape
  n_edges = src.shape[0]
  nw = NC * NS
  e_per_w = n_edges // nw
  nb = NB
  ch = CH
  n_chunks = e_per_w // ch
  assert e_per_w * nw == n_edges and n_chunks * ch == e_per_w
  assert n_chunks >= 2 * nb + 2
  big = (n_nodes // NS) // 8 * 8
  tail = n_nodes - NS * big
  assert 0 < tail <= big

  mesh = plsc.VectorSubcoreMesh(core_axis_name="c", subcore_axis_name="s")
  out_type = [jax.ShapeDtypeStruct((NC, n_nodes, n_feat), jnp.float32)]
  inputs = [x, src, dst, jnp.zeros((big, n_feat), jnp.float32)]
  scratch = [
      pltpu.VMEM((e_per_w,), jnp.int32),
      pltpu.VMEM((e_per_w,), jnp.int32),
  ] + [pltpu.VMEM((ch, n_feat), jnp.float32) for _ in range(nb)]
  if with_counts:
    out_type.append(jax.ShapeDtypeStruct((NC, n_nodes, n_feat), jnp.float32))
    inputs.append(jnp.ones((ch, n_feat), jnp.float32))
  scratch.append(pltpu.VMEM_SHARED((n_nodes, n_feat), jnp.float32))
  scratch += [pltpu.SemaphoreType.DMA for _ in range(2 * nb)]

  body = functools.partial(_segsum_body, with_counts, nb, ch, n_nodes, n_feat,
                           e_per_w, n_chunks, big)
  fn = pl.kernel(body, out_type=out_type, mesh=mesh, scratch_types=scratch)
  return fn(*inputs)


def _dense_body(relu, p_ref, c_ref, x_ref, wl_ref, b_ref, wr_ref, o_ref):
  ssum = p_ref[0] + p_ref[1]
  cnt = c_ref[0][:, 0:1] + c_ref[1][:, 0:1]
  mean = ssum / jnp.maximum(cnt, 1.0)
  acc = lax.dot_general(mean, wl_ref[...], (((1,), (1,)), ((), ())),
                        preferred_element_type=jnp.float32)
  acc = acc + lax.dot_general(x_ref[...], wr_ref[...], (((1,), (1,)), ((), ())),
                              preferred_element_type=jnp.float32)
  acc = acc + b_ref[...]
  o_ref[...] = jnp.maximum(acc, 0.0) if relu else acc


def _dense(p, cpart, x, wl, bl, wr, relu):
  n, f = x.shape
  blk = 1000
  grid = (n // blk,)
  body = functools.partial(_dense_body, relu)
  return pl.pallas_call(
      body,
      grid=grid,
      in_specs=[
          pl.BlockSpec((NC, blk, f), lambda i: (0, i, 0)),
          pl.BlockSpec((NC, blk, f), lambda i: (0, i, 0)),
          pl.BlockSpec((blk, f), lambda i: (i, 0)),
          pl.BlockSpec((f, f), lambda i: (0, 0)),
          pl.BlockSpec((1, f), lambda i: (0, 0)),
          pl.BlockSpec((f, f), lambda i: (0, 0)),
      ],
      out_specs=pl.BlockSpec((blk, f), lambda i: (i, 0)),
      out_shape=jax.ShapeDtypeStruct((n, f), jnp.float32),
  )(p, cpart, x, wl, bl.reshape(1, f), wr)


def kernel(x, edge_index, lin1_W, lin1_b, c1_Wl, c1_bl, c1_Wr,
           c2_Wl, c2_bl, c2_Wr):
  e32 = edge_index.astype(jnp.int32)
  src = e32[0]
  dst = e32[1]

  p1, cpart = _segsum(x, src, dst, True)
  h = _dense(p1, cpart, x, c1_Wl, c1_bl, c1_Wr, True)
  (p2,) = _segsum(h, src, dst, False)
  out = _dense(p2, cpart, h, c2_Wl, c2_bl, c2_Wr, False)
  return out
